# baseline (speedup 1.0000x reference)
import functools

import jax
import jax.numpy as jnp
from jax import lax
from jax.experimental import pallas as pl
from jax.experimental.pallas import tpu as pltpu
from jax.experimental.pallas import tpu_sc as plsc

N = 10000
E = 320000
D = 128
R = 4
NLAYERS = 2

NC = 2
NS = 16
NW = NC * NS
EPW = E // NW
CH = 128
NCHUNK = 81
EPT_PAD = NCHUNK * CH
NGRP = EPW // 16
NGRP_PAD = EPT_PAD // 16
CPAD = 40960
CNT_PER_TILE = CPAD // NS
PADCNT = R * N
NPAD = 10240
PADROW = N
ROWS_PER_TILE = NPAD // NS
ZROWS = 128

_MESH = plsc.VectorSubcoreMesh(
    core_axis_name="c", subcore_axis_name="s", num_cores=NC, num_subcores=NS)


@functools.partial(
    pl.kernel,
    out_type=(
        jax.ShapeDtypeStruct((NW * EPT_PAD,), jnp.int32),
        jax.ShapeDtypeStruct((NW * EPT_PAD,), jnp.int32),
        jax.ShapeDtypeStruct((NW, NCHUNK, CH), jnp.int32),
        jax.ShapeDtypeStruct((NC, CPAD), jnp.float32),
    ),
    mesh=_MESH,
    scratch_types=[
        pltpu.VMEM((EPW,), jnp.int32),
        pltpu.VMEM((EPW,), jnp.int32),
        pltpu.VMEM((EPW,), jnp.int32),
        pltpu.VMEM((EPT_PAD,), jnp.int32),
        pltpu.VMEM((EPT_PAD,), jnp.int32),
        pltpu.VMEM((NCHUNK, CH), jnp.int32),
        pltpu.VMEM((CH,), jnp.float32),
        pltpu.VMEM((CNT_PER_TILE,), jnp.float32),
        pltpu.VMEM_SHARED((CPAD,), jnp.float32),
        pltpu.SemaphoreType.DMA,
    ],
)
def _sc_counts(src_hbm, dst_hbm, typ_hbm, g_out, d_out, c_out, cnt_hbm,
               sv, dv, tv, gb, db, cb, ones, zb, cnt_sh, sem):
    c = lax.axis_index("c")
    s = lax.axis_index("s")
    wid = c * NS + s

    zero16 = jnp.zeros((16,), jnp.float32)
    one16 = jnp.ones((16,), jnp.float32)

    def _zb_body(i, carry):
        zb[pl.ds(i * 16, 16)] = zero16
        return carry
    lax.fori_loop(0, CNT_PER_TILE // 16, _zb_body, 0)
    for j in range(CH // 16):
        ones[pl.ds(j * 16, 16)] = one16

    pltpu.sync_copy(zb, cnt_sh.at[pl.ds(s * CNT_PER_TILE, CNT_PER_TILE)])

    base = wid * EPW
    pltpu.sync_copy(src_hbm.at[pl.ds(base, EPW)], sv)
    pltpu.sync_copy(dst_hbm.at[pl.ds(base, EPW)], dv)
    pltpu.sync_copy(typ_hbm.at[pl.ds(base, EPW)], tv)

    def _grp(i, carry):
        chunk = i // (CH // 16)
        off = (i % (CH // 16)) * 16
        sl16 = pl.ds(i * 16, 16)
        t_n = tv[sl16] * N
        dvec = dv[sl16]
        gb[sl16] = t_n + sv[sl16]
        db[sl16] = dvec
        cb[chunk, pl.ds(off, 16)] = t_n + dvec
        return carry
    lax.fori_loop(0, NGRP, _grp, 0)

    padrow16 = jnp.full((16,), PADROW, jnp.int32)
    padcnt16 = jnp.full((16,), PADCNT, jnp.int32)
    zero16i = jnp.zeros((16,), jnp.int32)

    def _pad(i, carry):
        chunk = i // (CH // 16)
        off = (i % (CH // 16)) * 16
        sl16 = pl.ds(i * 16, 16)
        gb[sl16] = zero16i
        db[sl16] = padrow16
        cb[chunk, pl.ds(off, 16)] = padcnt16
        return carry
    lax.fori_loop(NGRP, NGRP_PAD, _pad, 0)

    pltpu.sync_copy(gb, g_out.at[pl.ds(wid * EPT_PAD, EPT_PAD)])
    pltpu.sync_copy(db, d_out.at[pl.ds(wid * EPT_PAD, EPT_PAD)])
    pltpu.sync_copy(cb, c_out.at[wid])

    plsc.subcore_barrier()

    descs = []
    for i in range(NCHUNK):
        descs.append(
            pltpu.async_copy(ones, cnt_sh.at[cb.at[i]], sem, add=True))
    for d in descs:
        d.wait()

    plsc.subcore_barrier()
    sl = pl.ds(s * CNT_PER_TILE, CNT_PER_TILE)
    pltpu.sync_copy(cnt_sh.at[sl], cnt_hbm.at[c, sl])


@functools.partial(
    pl.kernel,
    out_type=jax.ShapeDtypeStruct((NW * EPT_PAD,), jnp.float32),
    mesh=_MESH,
    scratch_types=[
        pltpu.VMEM((NCHUNK, CH), jnp.int32),
        pltpu.VMEM((EPT_PAD,), jnp.float32),
        pltpu.SemaphoreType.DMA,
    ],
)
def _sc_scales(cidx_hbm, inv_hbm, s_out, cb, sb, sem):
    c = lax.axis_index("c")
    s = lax.axis_index("s")
    wid = c * NS + s
    pltpu.sync_copy(cidx_hbm.at[wid], cb)
    descs = []
    for i in range(NCHUNK):
        descs.append(pltpu.async_copy(
            inv_hbm.at[cb.at[i]], sb.at[pl.ds(i * CH, CH)], sem))
    for d in descs:
        d.wait()
    pltpu.sync_copy(sb, s_out.at[pl.ds(wid * EPT_PAD, EPT_PAD)])


AG_CH = 72
AG_NCHUNK = EPT_PAD // AG_CH
NSLOT = 2
ISLOT = 4
UNROLL = 4


@functools.partial(
    pl.kernel,
    out_type=jax.ShapeDtypeStruct((NC, NPAD, D), jnp.float32),
    mesh=_MESH,
    scratch_types=[
        pltpu.VMEM((ISLOT, AG_CH), jnp.int32),
        pltpu.VMEM((ISLOT, AG_CH), jnp.int32),
        pltpu.VMEM((ISLOT, AG_CH), jnp.float32),
        pltpu.VMEM((NSLOT, AG_CH, D), jnp.float32),
        pltpu.VMEM_SHARED((NPAD, D), jnp.float32),
        pltpu.SemaphoreType.DMA((ISLOT,)),
        pltpu.SemaphoreType.DMA((NSLOT,)),
        pltpu.SemaphoreType.DMA((NSLOT,)),
    ],
)
def _sc_aggregate(y_hbm, g_hbm, d_hbm, s_hbm, part_hbm,
                  gring, dring, sring, rows, acc, si, sg, ssc):
    c = lax.axis_index("c")
    s = lax.axis_index("s")
    wid = c * NS + s
    ebase = wid * EPT_PAD

    zero16 = jnp.zeros((16,), jnp.float32)

    def _zb_body(i, carry):
        r = i // (D // 16)
        j = lax.rem(i, D // 16)
        rows[0, r, pl.ds(j * 16, 16)] = zero16
        return carry
    lax.fori_loop(0, AG_CH * (D // 16), _zb_body, 0)
    for k in range(ROWS_PER_TILE // 64):
        pltpu.sync_copy(
            rows.at[0, pl.ds(0, 64)],
            acc.at[pl.ds(s * ROWS_PER_TILE + k * 64, 64)])

    def _idx_descs(i, t):
        off = pl.ds(ebase + i * AG_CH, AG_CH)
        return (
            pltpu.make_async_copy(g_hbm.at[off], gring.at[t], si.at[t]),
            pltpu.make_async_copy(d_hbm.at[off], dring.at[t], si.at[t]),
            pltpu.make_async_copy(s_hbm.at[off], sring.at[t], si.at[t]),
        )

    def _issue_idx(i, t):
        for dsc in _idx_descs(i, t):
            dsc.start()

    def _wait_idx(i, t):
        for dsc in _idx_descs(i, t):
            dsc.wait()

    def _gather_desc(i, t, b):
        return pltpu.make_async_copy(
            y_hbm.at[gring.at[t]], rows.at[b], sg.at[b])

    def _scatter_desc(i, t, b):
        return pltpu.make_async_copy(
            rows.at[b], acc.at[dring.at[t]], ssc.at[b])

    _issue_idx(0, 0)
    _issue_idx(1, 1)
    _wait_idx(0, 0)
    _gather_desc(0, 0, 0).start()

    plsc.subcore_barrier()

    def _chunk_step(i, b, t):
        _gather_desc(i, t, b).wait()

        @pl.when(i >= 1)
        def _():
            _scatter_desc(i - 1, (t - 1) % ISLOT, 1 - b).wait()

        @pl.when(i + 2 < AG_NCHUNK)
        def _():
            _issue_idx(i + 2, (t + 2) % ISLOT)

        @pl.when(i + 1 < AG_NCHUNK)
        def _():
            _wait_idx(i + 1, (t + 1) % ISLOT)
            _gather_desc(i + 1, (t + 1) % ISLOT, 1 - b).start()

        def _scale(k, icarry):
            svec = sring[t, pl.ds(k * 16, 16)]
            for j in range(16):
                sc = svec[j]
                e = k * 16 + j
                for m in range(D // 16):
                    sl = pl.ds(m * 16, 16)
                    rows[b, e, sl] = rows[b, e, sl] * sc
            return icarry
        lax.fori_loop(0, AG_CH // 16, _scale, 0)

        pltpu.async_copy(rows.at[b], acc.at[dring.at[t]], ssc.at[b], add=True)

    def _quad(q, carry):
        for u in range(UNROLL):
            i = q * UNROLL + u
            _chunk_step(i, u % NSLOT, u % ISLOT)
        return carry
    lax.fori_loop(0, AG_NCHUNK // UNROLL, _quad, 0)

    last = AG_NCHUNK - 1
    _scatter_desc(last, last % ISLOT, last % NSLOT).wait()

    plsc.subcore_barrier()
    for k in range(ROWS_PER_TILE // 64):
        sl = pl.ds(s * ROWS_PER_TILE + k * 64, 64)
        pltpu.sync_copy(acc.at[sl], part_hbm.at[c, sl])


def _inv_body(cnt_ref, inv_ref):
    total = cnt_ref[0] + cnt_ref[1]
    rows128 = CPAD // 128
    idx = (lax.broadcasted_iota(jnp.int32, (rows128, 128), 0) * 128
           + lax.broadcasted_iota(jnp.int32, (rows128, 128), 1))
    inv = 1.0 / jnp.maximum(total, 1.0)
    inv_ref[...] = jnp.where(idx < R * N, inv, 0.0)


def _tc_inv_counts(cnt):
    cnt2 = cnt.reshape(NC, CPAD // 128, 128)
    inv = pl.pallas_call(
        _inv_body,
        out_shape=jax.ShapeDtypeStruct((CPAD // 128, 128), jnp.float32),
    )(cnt2)
    return inv.reshape(CPAD)


_BN = 1000


def _transform_body(h_ref, w_ref, root_ref, bias_ref, y_ref, base_ref):
    h = h_ref[...]
    base_ref[...] = jnp.dot(h, root_ref[...],
                            preferred_element_type=jnp.float32) + bias_ref[...]
    for r in range(R):
        y_ref[r] = jnp.dot(h, w_ref[r], preferred_element_type=jnp.float32)


def _tc_transform(h, w, root, bias):
    return pl.pallas_call(
        _transform_body,
        grid=(N // _BN,),
        in_specs=[
            pl.BlockSpec((_BN, D), lambda i: (i, 0)),
            pl.BlockSpec((R, D, D), lambda i: (0, 0, 0)),
            pl.BlockSpec((D, D), lambda i: (0, 0)),
            pl.BlockSpec((1, D), lambda i: (0, 0)),
        ],
        out_specs=[
            pl.BlockSpec((R, _BN, D), lambda i: (0, i, 0)),
            pl.BlockSpec((_BN, D), lambda i: (i, 0)),
        ],
        out_shape=[
            jax.ShapeDtypeStruct((R, N, D), jnp.float32),
            jax.ShapeDtypeStruct((N, D), jnp.float32),
        ],
    )(h, w, root, bias.reshape(1, D))


def _combine_body(base_ref, part_ref, out_ref):
    out_ref[...] = jnp.maximum(base_ref[...] + part_ref[0] + part_ref[1], 0.0)


def _tc_combine(base, parts):
    return pl.pallas_call(
        _combine_body,
        grid=(N // _BN,),
        in_specs=[
            pl.BlockSpec((_BN, D), lambda i: (i, 0)),
            pl.BlockSpec((NC, _BN, D), lambda i: (0, i, 0)),
        ],
        out_specs=pl.BlockSpec((_BN, D), lambda i: (i, 0)),
        out_shape=jax.ShapeDtypeStruct((N, D), jnp.float32),
    )(base, parts)


@jax.jit
def kernel(x, edge_index, edge_type, weights, roots, biases):
    src = edge_index[0].astype(jnp.int32)
    dst = edge_index[1].astype(jnp.int32)
    typ = edge_type.astype(jnp.int32)

    gidx, didx, cidx, cnt = _sc_counts(src, dst, typ)
    inv = _tc_inv_counts(cnt)
    scales = _sc_scales(cidx, inv)

    h = x
    for l in range(NLAYERS):
        y, base = _tc_transform(h, weights[l], roots[l], biases[l])
        parts = _sc_aggregate(y.reshape(R * N, D), gidx, didx, scales)
        h = _tc_combine(base, parts)
    return h


# speedup vs baseline: 1.3902x; 1.3902x over previous
import functools

import jax
import jax.numpy as jnp
from jax import lax
from jax.experimental import pallas as pl
from jax.experimental.pallas import tpu as pltpu
from jax.experimental.pallas import tpu_sc as plsc

N = 10000
E = 320000
D = 128
R = 4
NLAYERS = 2

NC = 2
NS = 16
NW = NC * NS
EPW = E // NW
CH = 128
NCHUNK = 81
EPT_PAD = NCHUNK * CH
NGRP = EPW // 16
NGRP_PAD = EPT_PAD // 16
CPAD = 40960
CNT_PER_TILE = CPAD // NS
PADCNT = R * N
NPAD = 10240
PADROW = N
ROWS_PER_TILE = NPAD // NS
ZROWS = 128

_MESH = plsc.VectorSubcoreMesh(
    core_axis_name="c", subcore_axis_name="s", num_cores=NC, num_subcores=NS)


@functools.partial(
    pl.kernel,
    out_type=(
        jax.ShapeDtypeStruct((NW * EPT_PAD,), jnp.int32),
        jax.ShapeDtypeStruct((NW * EPT_PAD,), jnp.int32),
        jax.ShapeDtypeStruct((NW, NCHUNK, CH), jnp.int32),
        jax.ShapeDtypeStruct((NC, CPAD), jnp.float32),
    ),
    mesh=_MESH,
    scratch_types=[
        pltpu.VMEM((EPW,), jnp.int32),
        pltpu.VMEM((EPW,), jnp.int32),
        pltpu.VMEM((EPW,), jnp.int32),
        pltpu.VMEM((EPT_PAD,), jnp.int32),
        pltpu.VMEM((EPT_PAD,), jnp.int32),
        pltpu.VMEM((NCHUNK, CH), jnp.int32),
        pltpu.VMEM((CH,), jnp.float32),
        pltpu.VMEM((CNT_PER_TILE,), jnp.float32),
        pltpu.VMEM_SHARED((CPAD,), jnp.float32),
        pltpu.SemaphoreType.DMA,
    ],
)
def _sc_counts(src_hbm, dst_hbm, typ_hbm, g_out, d_out, c_out, cnt_hbm,
               sv, dv, tv, gb, db, cb, ones, zb, cnt_sh, sem):
    c = lax.axis_index("c")
    s = lax.axis_index("s")
    wid = c * NS + s

    zero16 = jnp.zeros((16,), jnp.float32)
    one16 = jnp.ones((16,), jnp.float32)

    def _zb_body(i, carry):
        zb[pl.ds(i * 16, 16)] = zero16
        return carry
    lax.fori_loop(0, CNT_PER_TILE // 16, _zb_body, 0)
    for j in range(CH // 16):
        ones[pl.ds(j * 16, 16)] = one16

    pltpu.sync_copy(zb, cnt_sh.at[pl.ds(s * CNT_PER_TILE, CNT_PER_TILE)])

    base = wid * EPW
    pltpu.sync_copy(src_hbm.at[pl.ds(base, EPW)], sv)
    pltpu.sync_copy(dst_hbm.at[pl.ds(base, EPW)], dv)
    pltpu.sync_copy(typ_hbm.at[pl.ds(base, EPW)], tv)

    def _grp(i, carry):
        chunk = i // (CH // 16)
        off = (i % (CH // 16)) * 16
        sl16 = pl.ds(i * 16, 16)
        t_n = tv[sl16] * N
        dvec = dv[sl16]
        gb[sl16] = t_n + sv[sl16]
        db[sl16] = dvec
        cb[chunk, pl.ds(off, 16)] = t_n + dvec
        return carry
    lax.fori_loop(0, NGRP, _grp, 0)

    padrow16 = jnp.full((16,), PADROW, jnp.int32)
    padcnt16 = jnp.full((16,), PADCNT, jnp.int32)
    zero16i = jnp.zeros((16,), jnp.int32)

    def _pad(i, carry):
        chunk = i // (CH // 16)
        off = (i % (CH // 16)) * 16
        sl16 = pl.ds(i * 16, 16)
        gb[sl16] = zero16i
        db[sl16] = padrow16
        cb[chunk, pl.ds(off, 16)] = padcnt16
        return carry
    lax.fori_loop(NGRP, NGRP_PAD, _pad, 0)

    pltpu.sync_copy(gb, g_out.at[pl.ds(wid * EPT_PAD, EPT_PAD)])
    pltpu.sync_copy(db, d_out.at[pl.ds(wid * EPT_PAD, EPT_PAD)])
    pltpu.sync_copy(cb, c_out.at[wid])

    plsc.subcore_barrier()

    descs = []
    for i in range(NCHUNK):
        descs.append(
            pltpu.async_copy(ones, cnt_sh.at[cb.at[i]], sem, add=True))
    for d in descs:
        d.wait()

    plsc.subcore_barrier()
    sl = pl.ds(s * CNT_PER_TILE, CNT_PER_TILE)
    pltpu.sync_copy(cnt_sh.at[sl], cnt_hbm.at[c, sl])


@functools.partial(
    pl.kernel,
    out_type=jax.ShapeDtypeStruct((NW * EPT_PAD,), jnp.float32),
    mesh=_MESH,
    scratch_types=[
        pltpu.VMEM((NCHUNK, CH), jnp.int32),
        pltpu.VMEM((EPT_PAD,), jnp.float32),
        pltpu.SemaphoreType.DMA,
    ],
)
def _sc_scales(cidx_hbm, inv_hbm, s_out, cb, sb, sem):
    c = lax.axis_index("c")
    s = lax.axis_index("s")
    wid = c * NS + s
    pltpu.sync_copy(cidx_hbm.at[wid], cb)
    descs = []
    for i in range(NCHUNK):
        descs.append(pltpu.async_copy(
            inv_hbm.at[cb.at[i]], sb.at[pl.ds(i * CH, CH)], sem))
    for d in descs:
        d.wait()
    pltpu.sync_copy(sb, s_out.at[pl.ds(wid * EPT_PAD, EPT_PAD)])


AG_CH = 80
AG_NCHUNK = EPW // AG_CH


@functools.partial(
    pl.kernel,
    out_type=jax.ShapeDtypeStruct((NC, NPAD, D), jnp.float32),
    mesh=_MESH,
    scratch_types=[
        pltpu.VMEM((AG_CH,), jnp.int32),
        pltpu.VMEM((AG_CH,), jnp.int32),
        pltpu.VMEM((AG_CH,), jnp.float32),
        pltpu.VMEM((AG_CH, D), jnp.float32),
        pltpu.VMEM_SHARED((NPAD, D), jnp.float32),
        pltpu.SemaphoreType.DMA,
    ],
)
def _sc_aggregate(y_hbm, g_hbm, d_hbm, s_hbm, part_hbm,
                  gv, dv, sv, rows, acc, sem):
    c = lax.axis_index("c")
    s = lax.axis_index("s")
    wid = c * NS + s
    ebase = wid * EPT_PAD

    zero16 = jnp.zeros((16,), jnp.float32)

    def _zb_body(i, carry):
        r = i // (D // 16)
        j = lax.rem(i, D // 16)
        rows[r, pl.ds(j * 16, 16)] = zero16
        return carry
    lax.fori_loop(0, AG_CH * (D // 16), _zb_body, 0)
    for k in range(ROWS_PER_TILE // AG_CH):
        pltpu.sync_copy(
            rows, acc.at[pl.ds(s * ROWS_PER_TILE + k * AG_CH, AG_CH)])
    plsc.subcore_barrier()

    def _chunk(i, carry):
        off = pl.ds(ebase + i * AG_CH, AG_CH)
        pltpu.sync_copy(g_hbm.at[off], gv)
        pltpu.sync_copy(d_hbm.at[off], dv)
        pltpu.sync_copy(s_hbm.at[off], sv)
        pltpu.async_copy(y_hbm.at[gv], rows, sem).wait()

        def _scale(k, icarry):
            svec = sv[pl.ds(k * 16, 16)]
            for j in range(16):
                sc = svec[j]
                e = k * 16 + j
                for m in range(D // 16):
                    sl = pl.ds(m * 16, 16)
                    rows[e, sl] = rows[e, sl] * sc
            return icarry
        lax.fori_loop(0, AG_CH // 16, _scale, 0)

        pltpu.sync_copy(rows, acc.at[dv], add=True)
        return carry
    lax.fori_loop(0, AG_NCHUNK, _chunk, 0)

    plsc.subcore_barrier()
    for k in range(ROWS_PER_TILE // AG_CH):
        sl = pl.ds(s * ROWS_PER_TILE + k * AG_CH, AG_CH)
        pltpu.sync_copy(acc.at[sl], part_hbm.at[c, sl])


def _inv_body(cnt_ref, inv_ref):
    total = cnt_ref[0] + cnt_ref[1]
    rows128 = CPAD // 128
    idx = (lax.broadcasted_iota(jnp.int32, (rows128, 128), 0) * 128
           + lax.broadcasted_iota(jnp.int32, (rows128, 128), 1))
    inv = 1.0 / jnp.maximum(total, 1.0)
    inv_ref[...] = jnp.where(idx < R * N, inv, 0.0)


def _tc_inv_counts(cnt):
    cnt2 = cnt.reshape(NC, CPAD // 128, 128)
    inv = pl.pallas_call(
        _inv_body,
        out_shape=jax.ShapeDtypeStruct((CPAD // 128, 128), jnp.float32),
    )(cnt2)
    return inv.reshape(CPAD)


_BN = 1000


def _transform_body(h_ref, w_ref, root_ref, bias_ref, y_ref, base_ref):
    h = h_ref[...]
    base_ref[...] = jnp.dot(h, root_ref[...],
                            preferred_element_type=jnp.float32) + bias_ref[...]
    for r in range(R):
        y_ref[r] = jnp.dot(h, w_ref[r], preferred_element_type=jnp.float32)


def _tc_transform(h, w, root, bias):
    return pl.pallas_call(
        _transform_body,
        grid=(N // _BN,),
        in_specs=[
            pl.BlockSpec((_BN, D), lambda i: (i, 0)),
            pl.BlockSpec((R, D, D), lambda i: (0, 0, 0)),
            pl.BlockSpec((D, D), lambda i: (0, 0)),
            pl.BlockSpec((1, D), lambda i: (0, 0)),
        ],
        out_specs=[
            pl.BlockSpec((R, _BN, D), lambda i: (0, i, 0)),
            pl.BlockSpec((_BN, D), lambda i: (i, 0)),
        ],
        out_shape=[
            jax.ShapeDtypeStruct((R, N, D), jnp.float32),
            jax.ShapeDtypeStruct((N, D), jnp.float32),
        ],
    )(h, w, root, bias.reshape(1, D))


def _combine_body(base_ref, part_ref, out_ref):
    out_ref[...] = jnp.maximum(base_ref[...] + part_ref[0] + part_ref[1], 0.0)


def _tc_combine(base, parts):
    return pl.pallas_call(
        _combine_body,
        grid=(N // _BN,),
        in_specs=[
            pl.BlockSpec((_BN, D), lambda i: (i, 0)),
            pl.BlockSpec((NC, _BN, D), lambda i: (0, i, 0)),
        ],
        out_specs=pl.BlockSpec((_BN, D), lambda i: (i, 0)),
        out_shape=jax.ShapeDtypeStruct((N, D), jnp.float32),
    )(base, parts)


@jax.jit
def kernel(x, edge_index, edge_type, weights, roots, biases):
    src = edge_index[0].astype(jnp.int32)
    dst = edge_index[1].astype(jnp.int32)
    typ = edge_type.astype(jnp.int32)

    gidx, didx, cidx, cnt = _sc_counts(src, dst, typ)
    inv = _tc_inv_counts(cnt)
    scales = _sc_scales(cidx, inv)

    h = x
    for l in range(NLAYERS):
        y, base = _tc_transform(h, weights[l], roots[l], biases[l])
        parts = _sc_aggregate(y.reshape(R * N, D), gidx, didx, scales)
        h = _tc_combine(base, parts)
    return h


# speedup vs baseline: 1.5477x; 1.1133x over previous
import functools

import jax
import jax.numpy as jnp
from jax import lax
from jax.experimental import pallas as pl
from jax.experimental.pallas import tpu as pltpu
from jax.experimental.pallas import tpu_sc as plsc

N = 10000
E = 320000
D = 128
R = 4
NLAYERS = 2

NC = 2
NS = 16
NW = NC * NS
EPW = E // NW
CH = 128
NCHUNK = 81
EPT_PAD = NCHUNK * CH
NGRP = EPW // 16
NGRP_PAD = EPT_PAD // 16
CPAD = 40960
CNT_PER_TILE = CPAD // NS
PADCNT = R * N
NPAD = 10240
PADROW = N
ROWS_PER_TILE = NPAD // NS
ZROWS = 128

_MESH = plsc.VectorSubcoreMesh(
    core_axis_name="c", subcore_axis_name="s", num_cores=NC, num_subcores=NS)


@functools.partial(
    pl.kernel,
    out_type=(
        jax.ShapeDtypeStruct((NW * EPT_PAD,), jnp.int32),
        jax.ShapeDtypeStruct((NW * EPT_PAD,), jnp.int32),
        jax.ShapeDtypeStruct((NW, NCHUNK, CH), jnp.int32),
        jax.ShapeDtypeStruct((NC, CPAD), jnp.float32),
    ),
    mesh=_MESH,
    scratch_types=[
        pltpu.VMEM((EPW,), jnp.int32),
        pltpu.VMEM((EPW,), jnp.int32),
        pltpu.VMEM((EPW,), jnp.int32),
        pltpu.VMEM((EPT_PAD,), jnp.int32),
        pltpu.VMEM((EPT_PAD,), jnp.int32),
        pltpu.VMEM((NCHUNK, CH), jnp.int32),
        pltpu.VMEM((CH,), jnp.float32),
        pltpu.VMEM((CNT_PER_TILE,), jnp.float32),
        pltpu.VMEM_SHARED((CPAD,), jnp.float32),
        pltpu.SemaphoreType.DMA,
    ],
)
def _sc_counts(src_hbm, dst_hbm, typ_hbm, g_out, d_out, c_out, cnt_hbm,
               sv, dv, tv, gb, db, cb, ones, zb, cnt_sh, sem):
    c = lax.axis_index("c")
    s = lax.axis_index("s")
    wid = c * NS + s

    zero16 = jnp.zeros((16,), jnp.float32)
    one16 = jnp.ones((16,), jnp.float32)

    def _zb_body(i, carry):
        zb[pl.ds(i * 16, 16)] = zero16
        return carry
    lax.fori_loop(0, CNT_PER_TILE // 16, _zb_body, 0)
    for j in range(CH // 16):
        ones[pl.ds(j * 16, 16)] = one16

    pltpu.sync_copy(zb, cnt_sh.at[pl.ds(s * CNT_PER_TILE, CNT_PER_TILE)])

    base = wid * EPW
    pltpu.sync_copy(src_hbm.at[pl.ds(base, EPW)], sv)
    pltpu.sync_copy(dst_hbm.at[pl.ds(base, EPW)], dv)
    pltpu.sync_copy(typ_hbm.at[pl.ds(base, EPW)], tv)

    def _grp(i, carry):
        chunk = i // (CH // 16)
        off = (i % (CH // 16)) * 16
        sl16 = pl.ds(i * 16, 16)
        t_n = tv[sl16] * N
        dvec = dv[sl16]
        gb[sl16] = t_n + sv[sl16]
        db[sl16] = dvec
        cb[chunk, pl.ds(off, 16)] = t_n + dvec
        return carry
    lax.fori_loop(0, NGRP, _grp, 0)

    padrow16 = jnp.full((16,), PADROW, jnp.int32)
    padcnt16 = jnp.full((16,), PADCNT, jnp.int32)
    zero16i = jnp.zeros((16,), jnp.int32)

    def _pad(i, carry):
        chunk = i // (CH // 16)
        off = (i % (CH // 16)) * 16
        sl16 = pl.ds(i * 16, 16)
        gb[sl16] = zero16i
        db[sl16] = padrow16
        cb[chunk, pl.ds(off, 16)] = padcnt16
        return carry
    lax.fori_loop(NGRP, NGRP_PAD, _pad, 0)

    pltpu.sync_copy(gb, g_out.at[pl.ds(wid * EPT_PAD, EPT_PAD)])
    pltpu.sync_copy(db, d_out.at[pl.ds(wid * EPT_PAD, EPT_PAD)])
    pltpu.sync_copy(cb, c_out.at[wid])

    plsc.subcore_barrier()

    descs = []
    for i in range(NCHUNK):
        descs.append(
            pltpu.async_copy(ones, cnt_sh.at[cb.at[i]], sem, add=True))
    for d in descs:
        d.wait()

    plsc.subcore_barrier()
    sl = pl.ds(s * CNT_PER_TILE, CNT_PER_TILE)
    pltpu.sync_copy(cnt_sh.at[sl], cnt_hbm.at[c, sl])


@functools.partial(
    pl.kernel,
    out_type=jax.ShapeDtypeStruct((NW * EPT_PAD,), jnp.float32),
    mesh=_MESH,
    scratch_types=[
        pltpu.VMEM((NCHUNK, CH), jnp.int32),
        pltpu.VMEM((EPT_PAD,), jnp.float32),
        pltpu.SemaphoreType.DMA,
    ],
)
def _sc_scales(cidx_hbm, inv_hbm, s_out, cb, sb, sem):
    c = lax.axis_index("c")
    s = lax.axis_index("s")
    wid = c * NS + s
    pltpu.sync_copy(cidx_hbm.at[wid], cb)
    descs = []
    for i in range(NCHUNK):
        descs.append(pltpu.async_copy(
            inv_hbm.at[cb.at[i]], sb.at[pl.ds(i * CH, CH)], sem))
    for d in descs:
        d.wait()
    pltpu.sync_copy(sb, s_out.at[pl.ds(wid * EPT_PAD, EPT_PAD)])


AG_CH = 80
AG_NCHUNK = EPW // AG_CH


@functools.partial(
    pl.kernel,
    out_type=jax.ShapeDtypeStruct((NC, NPAD, D), jnp.float32),
    mesh=_MESH,
    scratch_types=[
        pltpu.VMEM((AG_CH,), jnp.int32),
        pltpu.VMEM((2, AG_CH), jnp.int32),
        pltpu.VMEM((AG_CH,), jnp.float32),
        pltpu.VMEM((2, AG_CH, D), jnp.float32),
        pltpu.VMEM_SHARED((NPAD, D), jnp.float32),
        pltpu.SemaphoreType.DMA,
        pltpu.SemaphoreType.DMA,
        pltpu.SemaphoreType.DMA,
    ],
)
def _sc_aggregate(y_hbm, g_hbm, d_hbm, s_hbm, part_hbm,
                  gv, dv, sv, rows, acc, sem, ss0, ss1):
    c = lax.axis_index("c")
    s = lax.axis_index("s")
    wid = c * NS + s
    ebase = wid * EPT_PAD
    ss = (ss0, ss1)

    zero16 = jnp.zeros((16,), jnp.float32)

    def _zb_body(i, carry):
        r = i // (D // 16)
        j = lax.rem(i, D // 16)
        rows[0, r, pl.ds(j * 16, 16)] = zero16
        return carry
    lax.fori_loop(0, AG_CH * (D // 16), _zb_body, 0)
    for k in range(ROWS_PER_TILE // AG_CH):
        pltpu.sync_copy(
            rows.at[0], acc.at[pl.ds(s * ROWS_PER_TILE + k * AG_CH, AG_CH)])
    plsc.subcore_barrier()

    def _scatter_desc(b):
        return pltpu.make_async_copy(rows.at[b], acc.at[dv.at[b]], ss[b])

    def _chunk(i, b, first):
        off = pl.ds(ebase + i * AG_CH, AG_CH)
        pltpu.sync_copy(g_hbm.at[off], gv)
        pltpu.sync_copy(s_hbm.at[off], sv)
        pltpu.sync_copy(d_hbm.at[off], dv.at[b])
        if first:
            @pl.when(i >= 1)
            def _():
                _scatter_desc(1 - b).wait()
        else:
            _scatter_desc(1 - b).wait()
        pltpu.async_copy(y_hbm.at[gv], rows.at[b], sem).wait()

        def _scale(k, icarry):
            svec = sv[pl.ds(k * 16, 16)]
            for j in range(16):
                sc = svec[j]
                e = k * 16 + j
                for m in range(D // 16):
                    sl = pl.ds(m * 16, 16)
                    rows[b, e, sl] = rows[b, e, sl] * sc
            return icarry
        lax.fori_loop(0, AG_CH // 16, _scale, 0)

        pltpu.async_copy(rows.at[b], acc.at[dv.at[b]], ss[b], add=True)

    def _pair(q, carry):
        i = q * 2
        _chunk(i, 0, True)
        _chunk(i + 1, 1, False)
        return carry
    lax.fori_loop(0, (AG_NCHUNK - 1) // 2, _pair, 0)

    _chunk(AG_NCHUNK - 1, 0, False)
    _scatter_desc(0).wait()

    plsc.subcore_barrier()
    for k in range(ROWS_PER_TILE // AG_CH):
        sl = pl.ds(s * ROWS_PER_TILE + k * AG_CH, AG_CH)
        pltpu.sync_copy(acc.at[sl], part_hbm.at[c, sl])


def _inv_body(cnt_ref, inv_ref):
    total = cnt_ref[0] + cnt_ref[1]
    rows128 = CPAD // 128
    idx = (lax.broadcasted_iota(jnp.int32, (rows128, 128), 0) * 128
           + lax.broadcasted_iota(jnp.int32, (rows128, 128), 1))
    inv = 1.0 / jnp.maximum(total, 1.0)
    inv_ref[...] = jnp.where(idx < R * N, inv, 0.0)


def _tc_inv_counts(cnt):
    cnt2 = cnt.reshape(NC, CPAD // 128, 128)
    inv = pl.pallas_call(
        _inv_body,
        out_shape=jax.ShapeDtypeStruct((CPAD // 128, 128), jnp.float32),
    )(cnt2)
    return inv.reshape(CPAD)


_BN = 1000


def _transform_body(h_ref, w_ref, root_ref, bias_ref, y_ref, base_ref):
    h = h_ref[...]
    base_ref[...] = jnp.dot(h, root_ref[...],
                            preferred_element_type=jnp.float32) + bias_ref[...]
    for r in range(R):
        y_ref[r] = jnp.dot(h, w_ref[r], preferred_element_type=jnp.float32)


def _tc_transform(h, w, root, bias):
    return pl.pallas_call(
        _transform_body,
        grid=(N // _BN,),
        in_specs=[
            pl.BlockSpec((_BN, D), lambda i: (i, 0)),
            pl.BlockSpec((R, D, D), lambda i: (0, 0, 0)),
            pl.BlockSpec((D, D), lambda i: (0, 0)),
            pl.BlockSpec((1, D), lambda i: (0, 0)),
        ],
        out_specs=[
            pl.BlockSpec((R, _BN, D), lambda i: (0, i, 0)),
            pl.BlockSpec((_BN, D), lambda i: (i, 0)),
        ],
        out_shape=[
            jax.ShapeDtypeStruct((R, N, D), jnp.float32),
            jax.ShapeDtypeStruct((N, D), jnp.float32),
        ],
    )(h, w, root, bias.reshape(1, D))


def _combine_body(base_ref, part_ref, out_ref):
    out_ref[...] = jnp.maximum(base_ref[...] + part_ref[0] + part_ref[1], 0.0)


def _tc_combine(base, parts):
    return pl.pallas_call(
        _combine_body,
        grid=(N // _BN,),
        in_specs=[
            pl.BlockSpec((_BN, D), lambda i: (i, 0)),
            pl.BlockSpec((NC, _BN, D), lambda i: (0, i, 0)),
        ],
        out_specs=pl.BlockSpec((_BN, D), lambda i: (i, 0)),
        out_shape=jax.ShapeDtypeStruct((N, D), jnp.float32),
    )(base, parts)


@jax.jit
def kernel(x, edge_index, edge_type, weights, roots, biases):
    src = edge_index[0].astype(jnp.int32)
    dst = edge_index[1].astype(jnp.int32)
    typ = edge_type.astype(jnp.int32)

    gidx, didx, cidx, cnt = _sc_counts(src, dst, typ)
    inv = _tc_inv_counts(cnt)
    scales = _sc_scales(cidx, inv)

    h = x
    for l in range(NLAYERS):
        y, base = _tc_transform(h, weights[l], roots[l], biases[l])
        parts = _sc_aggregate(y.reshape(R * N, D), gidx, didx, scales)
        h = _tc_combine(base, parts)
    return h


# speedup vs baseline: 2.0214x; 1.3061x over previous
import functools

import jax
import jax.numpy as jnp
from jax import lax
from jax.experimental import pallas as pl
from jax.experimental.pallas import tpu as pltpu
from jax.experimental.pallas import tpu_sc as plsc

N = 10000
E = 320000
D = 128
R = 4
NLAYERS = 2

NC = 2
NS = 16
NW = NC * NS
EPW = E // NW
CH = 128
NCHUNK = 81
EPT_PAD = NCHUNK * CH
NGRP = EPW // 16
NGRP_PAD = EPT_PAD // 16
CPAD = 40960
CNT_PER_TILE = CPAD // NS
PADCNT = R * N
NPAD = 10240
PADROW = N
ROWS_PER_TILE = NPAD // NS
ZROWS = 128

_MESH = plsc.VectorSubcoreMesh(
    core_axis_name="c", subcore_axis_name="s", num_cores=NC, num_subcores=NS)


@functools.partial(
    pl.kernel,
    out_type=(
        jax.ShapeDtypeStruct((NW * EPT_PAD,), jnp.int32),
        jax.ShapeDtypeStruct((NW * EPT_PAD,), jnp.int32),
        jax.ShapeDtypeStruct((NW, NCHUNK, CH), jnp.int32),
        jax.ShapeDtypeStruct((NC, CPAD), jnp.float32),
    ),
    mesh=_MESH,
    scratch_types=[
        pltpu.VMEM((EPW,), jnp.int32),
        pltpu.VMEM((EPW,), jnp.int32),
        pltpu.VMEM((EPW,), jnp.int32),
        pltpu.VMEM((EPT_PAD,), jnp.int32),
        pltpu.VMEM((EPT_PAD,), jnp.int32),
        pltpu.VMEM((NCHUNK, CH), jnp.int32),
        pltpu.VMEM((CH,), jnp.float32),
        pltpu.VMEM((CNT_PER_TILE,), jnp.float32),
        pltpu.VMEM_SHARED((CPAD,), jnp.float32),
        pltpu.SemaphoreType.DMA,
    ],
)
def _sc_counts(src_hbm, dst_hbm, typ_hbm, g_out, d_out, c_out, cnt_hbm,
               sv, dv, tv, gb, db, cb, ones, zb, cnt_sh, sem):
    c = lax.axis_index("c")
    s = lax.axis_index("s")
    wid = c * NS + s

    zero16 = jnp.zeros((16,), jnp.float32)
    one16 = jnp.ones((16,), jnp.float32)

    def _zb_body(i, carry):
        zb[pl.ds(i * 16, 16)] = zero16
        return carry
    lax.fori_loop(0, CNT_PER_TILE // 16, _zb_body, 0)
    for j in range(CH // 16):
        ones[pl.ds(j * 16, 16)] = one16

    pltpu.sync_copy(zb, cnt_sh.at[pl.ds(s * CNT_PER_TILE, CNT_PER_TILE)])

    base = wid * EPW
    pltpu.sync_copy(src_hbm.at[pl.ds(base, EPW)], sv)
    pltpu.sync_copy(dst_hbm.at[pl.ds(base, EPW)], dv)
    pltpu.sync_copy(typ_hbm.at[pl.ds(base, EPW)], tv)

    def _grp(i, carry):
        chunk = i // (CH // 16)
        off = (i % (CH // 16)) * 16
        sl16 = pl.ds(i * 16, 16)
        t_n = tv[sl16] * N
        dvec = dv[sl16]
        gb[sl16] = t_n + sv[sl16]
        db[sl16] = dvec
        cb[chunk, pl.ds(off, 16)] = t_n + dvec
        return carry
    lax.fori_loop(0, NGRP, _grp, 0)

    padrow16 = jnp.full((16,), PADROW, jnp.int32)
    padcnt16 = jnp.full((16,), PADCNT, jnp.int32)
    zero16i = jnp.zeros((16,), jnp.int32)

    def _pad(i, carry):
        chunk = i // (CH // 16)
        off = (i % (CH // 16)) * 16
        sl16 = pl.ds(i * 16, 16)
        gb[sl16] = zero16i
        db[sl16] = padrow16
        cb[chunk, pl.ds(off, 16)] = padcnt16
        return carry
    lax.fori_loop(NGRP, NGRP_PAD, _pad, 0)

    pltpu.sync_copy(gb, g_out.at[pl.ds(wid * EPT_PAD, EPT_PAD)])
    pltpu.sync_copy(db, d_out.at[pl.ds(wid * EPT_PAD, EPT_PAD)])
    pltpu.sync_copy(cb, c_out.at[wid])

    plsc.subcore_barrier()

    descs = []
    for i in range(NCHUNK):
        descs.append(
            pltpu.async_copy(ones, cnt_sh.at[cb.at[i]], sem, add=True))
    for d in descs:
        d.wait()

    plsc.subcore_barrier()
    sl = pl.ds(s * CNT_PER_TILE, CNT_PER_TILE)
    pltpu.sync_copy(cnt_sh.at[sl], cnt_hbm.at[c, sl])


@functools.partial(
    pl.kernel,
    out_type=jax.ShapeDtypeStruct((NW * EPT_PAD,), jnp.float32),
    mesh=_MESH,
    scratch_types=[
        pltpu.VMEM((NCHUNK, CH), jnp.int32),
        pltpu.VMEM((EPT_PAD,), jnp.float32),
        pltpu.SemaphoreType.DMA,
    ],
)
def _sc_scales(cidx_hbm, inv_hbm, s_out, cb, sb, sem):
    c = lax.axis_index("c")
    s = lax.axis_index("s")
    wid = c * NS + s
    pltpu.sync_copy(cidx_hbm.at[wid], cb)
    descs = []
    for i in range(NCHUNK):
        descs.append(pltpu.async_copy(
            inv_hbm.at[cb.at[i]], sb.at[pl.ds(i * CH, CH)], sem))
    for d in descs:
        d.wait()
    pltpu.sync_copy(sb, s_out.at[pl.ds(wid * EPT_PAD, EPT_PAD)])


AG_CH = 80
AG_NCHUNK = EPW // AG_CH


@functools.partial(
    pl.kernel,
    out_type=jax.ShapeDtypeStruct((NC, NPAD, D), jnp.float32),
    mesh=_MESH,
    scratch_types=[
        pltpu.VMEM((2, AG_CH), jnp.int32),
        pltpu.VMEM((2, AG_CH), jnp.int32),
        pltpu.VMEM((2, AG_CH), jnp.float32),
        pltpu.VMEM((2, AG_CH, D), jnp.float32),
        pltpu.VMEM_SHARED((NPAD, D), jnp.float32),
        pltpu.SemaphoreType.DMA,
        pltpu.SemaphoreType.DMA,
        pltpu.SemaphoreType.DMA,
        pltpu.SemaphoreType.DMA,
        pltpu.SemaphoreType.DMA,
    ],
)
def _sc_aggregate(y_hbm, g_hbm, d_hbm, s_hbm, part_hbm,
                  gv, dv, sv, rows, acc, sem, ss0, ss1, si0, si1):
    c = lax.axis_index("c")
    s = lax.axis_index("s")
    wid = c * NS + s
    ebase = wid * EPT_PAD
    ss = (ss0, ss1)
    si = (si0, si1)

    zero16 = jnp.zeros((16,), jnp.float32)

    def _zb_body(i, carry):
        r = i // (D // 16)
        j = lax.rem(i, D // 16)
        rows[0, r, pl.ds(j * 16, 16)] = zero16
        return carry
    lax.fori_loop(0, AG_CH * (D // 16), _zb_body, 0)
    for k in range(ROWS_PER_TILE // AG_CH):
        pltpu.sync_copy(
            rows.at[0], acc.at[pl.ds(s * ROWS_PER_TILE + k * AG_CH, AG_CH)])
    plsc.subcore_barrier()

    def _scatter_desc(b):
        return pltpu.make_async_copy(rows.at[b], acc.at[dv.at[b]], ss[b])

    def _idx_descs(i, b):
        off = pl.ds(ebase + i * AG_CH, AG_CH)
        return (
            pltpu.make_async_copy(g_hbm.at[off], gv.at[b], si[b]),
            pltpu.make_async_copy(d_hbm.at[off], dv.at[b], si[b]),
            pltpu.make_async_copy(s_hbm.at[off], sv.at[b], si[b]),
        )

    def _chunk(i, b, first):
        for dsc in _idx_descs(i, b):
            dsc.wait()
        if first:
            @pl.when(i >= 1)
            def _():
                _scatter_desc(1 - b).wait()
        else:
            _scatter_desc(1 - b).wait()

        @pl.when(i + 1 < AG_NCHUNK)
        def _():
            for dsc in _idx_descs(i + 1, 1 - b):
                dsc.start()

        pltpu.async_copy(y_hbm.at[gv.at[b]], rows.at[b], sem).wait()

        def _scale(k, icarry):
            svec = sv[b, pl.ds(k * 16, 16)]
            for j in range(16):
                sc = svec[j]
                e = k * 16 + j
                for m in range(D // 16):
                    sl = pl.ds(m * 16, 16)
                    rows[b, e, sl] = rows[b, e, sl] * sc
            return icarry
        lax.fori_loop(0, AG_CH // 16, _scale, 0)

        pltpu.async_copy(rows.at[b], acc.at[dv.at[b]], ss[b], add=True)

    for dsc in _idx_descs(0, 0):
        dsc.start()

    def _pair(q, carry):
        i = q * 2
        _chunk(i, 0, True)
        _chunk(i + 1, 1, False)
        return carry
    lax.fori_loop(0, (AG_NCHUNK - 1) // 2, _pair, 0)

    _chunk(AG_NCHUNK - 1, 0, False)
    _scatter_desc(0).wait()

    plsc.subcore_barrier()
    for k in range(ROWS_PER_TILE // AG_CH):
        sl = pl.ds(s * ROWS_PER_TILE + k * AG_CH, AG_CH)
        pltpu.sync_copy(acc.at[sl], part_hbm.at[c, sl])


def _inv_body(cnt_ref, inv_ref):
    total = cnt_ref[0] + cnt_ref[1]
    rows128 = CPAD // 128
    idx = (lax.broadcasted_iota(jnp.int32, (rows128, 128), 0) * 128
           + lax.broadcasted_iota(jnp.int32, (rows128, 128), 1))
    inv = 1.0 / jnp.maximum(total, 1.0)
    inv_ref[...] = jnp.where(idx < R * N, inv, 0.0)


def _tc_inv_counts(cnt):
    cnt2 = cnt.reshape(NC, CPAD // 128, 128)
    inv = pl.pallas_call(
        _inv_body,
        out_shape=jax.ShapeDtypeStruct((CPAD // 128, 128), jnp.float32),
    )(cnt2)
    return inv.reshape(CPAD)


_BN = 1000


def _transform_body(h_ref, w_ref, root_ref, bias_ref, y_ref, base_ref):
    h = h_ref[...]
    base_ref[...] = jnp.dot(h, root_ref[...],
                            preferred_element_type=jnp.float32) + bias_ref[...]
    for r in range(R):
        y_ref[r] = jnp.dot(h, w_ref[r], preferred_element_type=jnp.float32)


def _tc_transform(h, w, root, bias):
    return pl.pallas_call(
        _transform_body,
        grid=(N // _BN,),
        in_specs=[
            pl.BlockSpec((_BN, D), lambda i: (i, 0)),
            pl.BlockSpec((R, D, D), lambda i: (0, 0, 0)),
            pl.BlockSpec((D, D), lambda i: (0, 0)),
            pl.BlockSpec((1, D), lambda i: (0, 0)),
        ],
        out_specs=[
            pl.BlockSpec((R, _BN, D), lambda i: (0, i, 0)),
            pl.BlockSpec((_BN, D), lambda i: (i, 0)),
        ],
        out_shape=[
            jax.ShapeDtypeStruct((R, N, D), jnp.float32),
            jax.ShapeDtypeStruct((N, D), jnp.float32),
        ],
    )(h, w, root, bias.reshape(1, D))


def _combine_body(base_ref, part_ref, out_ref):
    out_ref[...] = jnp.maximum(base_ref[...] + part_ref[0] + part_ref[1], 0.0)


def _tc_combine(base, parts):
    return pl.pallas_call(
        _combine_body,
        grid=(N // _BN,),
        in_specs=[
            pl.BlockSpec((_BN, D), lambda i: (i, 0)),
            pl.BlockSpec((NC, _BN, D), lambda i: (0, i, 0)),
        ],
        out_specs=pl.BlockSpec((_BN, D), lambda i: (i, 0)),
        out_shape=jax.ShapeDtypeStruct((N, D), jnp.float32),
    )(base, parts)


@jax.jit
def kernel(x, edge_index, edge_type, weights, roots, biases):
    src = edge_index[0].astype(jnp.int32)
    dst = edge_index[1].astype(jnp.int32)
    typ = edge_type.astype(jnp.int32)

    gidx, didx, cidx, cnt = _sc_counts(src, dst, typ)
    inv = _tc_inv_counts(cnt)
    scales = _sc_scales(cidx, inv)

    h = x
    for l in range(NLAYERS):
        y, base = _tc_transform(h, weights[l], roots[l], biases[l])
        parts = _sc_aggregate(y.reshape(R * N, D), gidx, didx, scales)
        h = _tc_combine(base, parts)
    return h


# speedup vs baseline: 2.7905x; 1.3805x over previous
import functools

import jax
import jax.numpy as jnp
from jax import lax
from jax.experimental import pallas as pl
from jax.experimental.pallas import tpu as pltpu
from jax.experimental.pallas import tpu_sc as plsc

N = 10000
E = 320000
D = 128
R = 4
NLAYERS = 2

NC = 2
NS = 16
NW = NC * NS
EPW = E // NW
CH = 128
NCHUNK = 81
EPT_PAD = NCHUNK * CH
NGRP = EPW // 16
NGRP_PAD = EPT_PAD // 16
CPAD = 40960
CNT_PER_TILE = CPAD // NS
PADCNT = R * N
NPAD = 10240
PADROW = N
ROWS_PER_TILE = NPAD // NS
ZROWS = 128

_MESH = plsc.VectorSubcoreMesh(
    core_axis_name="c", subcore_axis_name="s", num_cores=NC, num_subcores=NS)


@functools.partial(
    pl.kernel,
    out_type=(
        jax.ShapeDtypeStruct((NW * EPT_PAD,), jnp.int32),
        jax.ShapeDtypeStruct((NW * EPT_PAD,), jnp.int32),
        jax.ShapeDtypeStruct((NW, NCHUNK, CH), jnp.int32),
        jax.ShapeDtypeStruct((NC, CPAD), jnp.float32),
    ),
    mesh=_MESH,
    scratch_types=[
        pltpu.VMEM((EPW,), jnp.int32),
        pltpu.VMEM((EPW,), jnp.int32),
        pltpu.VMEM((EPW,), jnp.int32),
        pltpu.VMEM((EPT_PAD,), jnp.int32),
        pltpu.VMEM((EPT_PAD,), jnp.int32),
        pltpu.VMEM((NCHUNK, CH), jnp.int32),
        pltpu.VMEM((CH,), jnp.float32),
        pltpu.VMEM((CNT_PER_TILE,), jnp.float32),
        pltpu.VMEM_SHARED((CPAD,), jnp.float32),
        pltpu.SemaphoreType.DMA,
    ],
)
def _sc_counts(src_hbm, dst_hbm, typ_hbm, g_out, d_out, c_out, cnt_hbm,
               sv, dv, tv, gb, db, cb, ones, zb, cnt_sh, sem):
    c = lax.axis_index("c")
    s = lax.axis_index("s")
    wid = c * NS + s

    zero16 = jnp.zeros((16,), jnp.float32)
    one16 = jnp.ones((16,), jnp.float32)

    def _zb_body(i, carry):
        zb[pl.ds(i * 16, 16)] = zero16
        return carry
    lax.fori_loop(0, CNT_PER_TILE // 16, _zb_body, 0)
    for j in range(CH // 16):
        ones[pl.ds(j * 16, 16)] = one16

    pltpu.sync_copy(zb, cnt_sh.at[pl.ds(s * CNT_PER_TILE, CNT_PER_TILE)])

    base = wid * EPW
    pltpu.sync_copy(src_hbm.at[pl.ds(base, EPW)], sv)
    pltpu.sync_copy(dst_hbm.at[pl.ds(base, EPW)], dv)
    pltpu.sync_copy(typ_hbm.at[pl.ds(base, EPW)], tv)

    def _grp(i, carry):
        chunk = i // (CH // 16)
        off = (i % (CH // 16)) * 16
        sl16 = pl.ds(i * 16, 16)
        t_n = tv[sl16] * N
        dvec = dv[sl16]
        gb[sl16] = t_n + sv[sl16]
        db[sl16] = dvec
        cb[chunk, pl.ds(off, 16)] = t_n + dvec
        return carry
    lax.fori_loop(0, NGRP, _grp, 0)

    padrow16 = jnp.full((16,), PADROW, jnp.int32)
    padcnt16 = jnp.full((16,), PADCNT, jnp.int32)
    zero16i = jnp.zeros((16,), jnp.int32)

    def _pad(i, carry):
        chunk = i // (CH // 16)
        off = (i % (CH // 16)) * 16
        sl16 = pl.ds(i * 16, 16)
        gb[sl16] = zero16i
        db[sl16] = padrow16
        cb[chunk, pl.ds(off, 16)] = padcnt16
        return carry
    lax.fori_loop(NGRP, NGRP_PAD, _pad, 0)

    pltpu.sync_copy(gb, g_out.at[pl.ds(wid * EPT_PAD, EPT_PAD)])
    pltpu.sync_copy(db, d_out.at[pl.ds(wid * EPT_PAD, EPT_PAD)])
    pltpu.sync_copy(cb, c_out.at[wid])

    plsc.subcore_barrier()

    descs = []
    for i in range(NCHUNK):
        descs.append(
            pltpu.async_copy(ones, cnt_sh.at[cb.at[i]], sem, add=True))
    for d in descs:
        d.wait()

    plsc.subcore_barrier()
    sl = pl.ds(s * CNT_PER_TILE, CNT_PER_TILE)
    pltpu.sync_copy(cnt_sh.at[sl], cnt_hbm.at[c, sl])


@functools.partial(
    pl.kernel,
    out_type=jax.ShapeDtypeStruct((NW * EPT_PAD,), jnp.float32),
    mesh=_MESH,
    scratch_types=[
        pltpu.VMEM((NCHUNK, CH), jnp.int32),
        pltpu.VMEM((EPT_PAD,), jnp.float32),
        pltpu.SemaphoreType.DMA,
    ],
)
def _sc_scales(cidx_hbm, inv_hbm, s_out, cb, sb, sem):
    c = lax.axis_index("c")
    s = lax.axis_index("s")
    wid = c * NS + s
    pltpu.sync_copy(cidx_hbm.at[wid], cb)
    descs = []
    for i in range(NCHUNK):
        descs.append(pltpu.async_copy(
            inv_hbm.at[cb.at[i]], sb.at[pl.ds(i * CH, CH)], sem))
    for d in descs:
        d.wait()
    pltpu.sync_copy(sb, s_out.at[pl.ds(wid * EPT_PAD, EPT_PAD)])


AG_CH = 80
AG_NCHUNK = EPW // AG_CH


@functools.partial(
    pl.kernel,
    out_type=jax.ShapeDtypeStruct((NC, NPAD, D), jnp.float32),
    mesh=_MESH,
    scratch_types=[
        pltpu.VMEM((3, AG_CH), jnp.int32),
        pltpu.VMEM((3, AG_CH), jnp.int32),
        pltpu.VMEM((3, AG_CH), jnp.float32),
        pltpu.VMEM((3, AG_CH, D), jnp.float32),
        pltpu.VMEM_SHARED((NPAD, D), jnp.float32),
        pltpu.SemaphoreType.DMA,
        pltpu.SemaphoreType.DMA,
        pltpu.SemaphoreType.DMA,
        pltpu.SemaphoreType.DMA,
        pltpu.SemaphoreType.DMA,
        pltpu.SemaphoreType.DMA,
        pltpu.SemaphoreType.DMA,
        pltpu.SemaphoreType.DMA,
        pltpu.SemaphoreType.DMA,
    ],
)
def _sc_aggregate(y_hbm, g_hbm, d_hbm, s_hbm, part_hbm,
                  gv, dv, sv, rows, acc,
                  sg0, sg1, sg2, ss0, ss1, ss2, si0, si1, si2):
    c = lax.axis_index("c")
    s = lax.axis_index("s")
    wid = c * NS + s
    ebase = wid * EPT_PAD
    sg = (sg0, sg1, sg2)
    ss = (ss0, ss1, ss2)
    si = (si0, si1, si2)

    zero16 = jnp.zeros((16,), jnp.float32)

    def _zb_body(i, carry):
        r = i // (D // 16)
        j = lax.rem(i, D // 16)
        rows[0, r, pl.ds(j * 16, 16)] = zero16
        return carry
    lax.fori_loop(0, AG_CH * (D // 16), _zb_body, 0)
    for k in range(ROWS_PER_TILE // AG_CH):
        pltpu.sync_copy(
            rows.at[0], acc.at[pl.ds(s * ROWS_PER_TILE + k * AG_CH, AG_CH)])
    plsc.subcore_barrier()

    def _idx_descs(i, p):
        off = pl.ds(ebase + i * AG_CH, AG_CH)
        return (
            pltpu.make_async_copy(g_hbm.at[off], gv.at[p], si[p]),
            pltpu.make_async_copy(d_hbm.at[off], dv.at[p], si[p]),
            pltpu.make_async_copy(s_hbm.at[off], sv.at[p], si[p]),
        )

    def _gather_desc(p):
        return pltpu.make_async_copy(y_hbm.at[gv.at[p]], rows.at[p], sg[p])

    def _scatter_desc(p):
        return pltpu.make_async_copy(rows.at[p], acc.at[dv.at[p]], ss[p])

    def _chunk(i, p, stage):
        _gather_desc(p).wait()
        if stage == 1:
            pass
        else:
            _scatter_desc((p - 1) % 3).wait()
        if stage in (0, 1, 2):
            @pl.when(i + 2 < AG_NCHUNK)
            def _():
                for dsc in _idx_descs(i + 2, (p + 2) % 3):
                    dsc.start()
        if stage != 3:
            for dsc in _idx_descs(i + 1, (p + 1) % 3):
                dsc.wait()
            _gather_desc((p + 1) % 3).start()

        def _scale(k, icarry):
            svec = sv[p, pl.ds(k * 16, 16)]
            for j in range(16):
                sc = svec[j]
                e = k * 16 + j
                for m in range(D // 16):
                    sl = pl.ds(m * 16, 16)
                    rows[p, e, sl] = rows[p, e, sl] * sc
            return icarry
        lax.fori_loop(0, AG_CH // 16, _scale, 0)

        pltpu.async_copy(rows.at[p], acc.at[dv.at[p]], ss[p], add=True)

    for dsc in _idx_descs(0, 0):
        dsc.start()
    for dsc in _idx_descs(1, 1):
        dsc.start()
    for dsc in _idx_descs(0, 0):
        dsc.wait()
    _gather_desc(0).start()

    _chunk(0, 0, 1)
    _chunk(1, 1, 2)

    def _tri(q, carry):
        i = 2 + q * 3
        _chunk(i, 2, 0)
        _chunk(i + 1, 0, 0)
        _chunk(i + 2, 1, 0)
        return carry
    lax.fori_loop(0, (AG_NCHUNK - 5) // 3, _tri, 0)

    _chunk(AG_NCHUNK - 3, (AG_NCHUNK - 3) % 3, 0)
    _chunk(AG_NCHUNK - 2, (AG_NCHUNK - 2) % 3, 0)
    _chunk(AG_NCHUNK - 1, (AG_NCHUNK - 1) % 3, 3)
    _scatter_desc((AG_NCHUNK - 1) % 3).wait()

    plsc.subcore_barrier()
    for k in range(ROWS_PER_TILE // AG_CH):
        sl = pl.ds(s * ROWS_PER_TILE + k * AG_CH, AG_CH)
        pltpu.sync_copy(acc.at[sl], part_hbm.at[c, sl])


def _inv_body(cnt_ref, inv_ref):
    total = cnt_ref[0] + cnt_ref[1]
    rows128 = CPAD // 128
    idx = (lax.broadcasted_iota(jnp.int32, (rows128, 128), 0) * 128
           + lax.broadcasted_iota(jnp.int32, (rows128, 128), 1))
    inv = 1.0 / jnp.maximum(total, 1.0)
    inv_ref[...] = jnp.where(idx < R * N, inv, 0.0)


def _tc_inv_counts(cnt):
    cnt2 = cnt.reshape(NC, CPAD // 128, 128)
    inv = pl.pallas_call(
        _inv_body,
        out_shape=jax.ShapeDtypeStruct((CPAD // 128, 128), jnp.float32),
    )(cnt2)
    return inv.reshape(CPAD)


_BN = 1000


def _transform_body(h_ref, w_ref, root_ref, bias_ref, y_ref, base_ref):
    h = h_ref[...]
    base_ref[...] = jnp.dot(h, root_ref[...],
                            preferred_element_type=jnp.float32) + bias_ref[...]
    for r in range(R):
        y_ref[r] = jnp.dot(h, w_ref[r], preferred_element_type=jnp.float32)


def _tc_transform(h, w, root, bias):
    return pl.pallas_call(
        _transform_body,
        grid=(N // _BN,),
        in_specs=[
            pl.BlockSpec((_BN, D), lambda i: (i, 0)),
            pl.BlockSpec((R, D, D), lambda i: (0, 0, 0)),
            pl.BlockSpec((D, D), lambda i: (0, 0)),
            pl.BlockSpec((1, D), lambda i: (0, 0)),
        ],
        out_specs=[
            pl.BlockSpec((R, _BN, D), lambda i: (0, i, 0)),
            pl.BlockSpec((_BN, D), lambda i: (i, 0)),
        ],
        out_shape=[
            jax.ShapeDtypeStruct((R, N, D), jnp.float32),
            jax.ShapeDtypeStruct((N, D), jnp.float32),
        ],
    )(h, w, root, bias.reshape(1, D))


def _combine_body(base_ref, part_ref, out_ref):
    out_ref[...] = jnp.maximum(base_ref[...] + part_ref[0] + part_ref[1], 0.0)


def _tc_combine(base, parts):
    return pl.pallas_call(
        _combine_body,
        grid=(N // _BN,),
        in_specs=[
            pl.BlockSpec((_BN, D), lambda i: (i, 0)),
            pl.BlockSpec((NC, _BN, D), lambda i: (0, i, 0)),
        ],
        out_specs=pl.BlockSpec((_BN, D), lambda i: (i, 0)),
        out_shape=jax.ShapeDtypeStruct((N, D), jnp.float32),
    )(base, parts)


@jax.jit
def kernel(x, edge_index, edge_type, weights, roots, biases):
    src = edge_index[0].astype(jnp.int32)
    dst = edge_index[1].astype(jnp.int32)
    typ = edge_type.astype(jnp.int32)

    gidx, didx, cidx, cnt = _sc_counts(src, dst, typ)
    inv = _tc_inv_counts(cnt)
    scales = _sc_scales(cidx, inv)

    h = x
    for l in range(NLAYERS):
        y, base = _tc_transform(h, weights[l], roots[l], biases[l])
        parts = _sc_aggregate(y.reshape(R * N, D), gidx, didx, scales)
        h = _tc_combine(base, parts)
    return h
